# Initial kernel scaffold; baseline (speedup 1.0000x reference)
import jax
import jax.numpy as jnp
from jax.experimental import pallas as pl


def kernel(x, edge_index, edge_attr, Ws, bs):
    raise NotImplementedError("write your pallas kernel here")


# speedup vs baseline: 3.4190x; 3.4190x over previous
import functools

import jax
import jax.numpy as jnp
from jax import lax
from jax.experimental import pallas as pl
from jax.experimental.pallas import tpu as pltpu
from jax.experimental.pallas import tpu_sc as plsc

NC = 2
NS = 16
LN = 16


def _mesh():
    return plsc.VectorSubcoreMesh(
        core_axis_name="c", subcore_axis_name="s", num_cores=NC, num_subcores=NS
    )


_SC_PARAMS = pltpu.CompilerParams(needs_layout_passes=False)


def _rsqrt_newton(x):
    i = lax.bitcast_convert_type(x, jnp.int32)
    y = lax.bitcast_convert_type(jnp.int32(0x5F3759DF) - (i >> 1), jnp.float32)
    for _ in range(3):
        y = y * (1.5 - 0.5 * x * y * y)
    return jnp.where(x > 0.0, y, 0.0)


def _gelu(x):
    return 0.5 * x * (1.0 + lax.erf(x * 0.7071067811865476))


def _norm_body(npad, erows, rowm, colm, wm, zf, seq, norm_out,
               colb_v, wb_v, deg_v, degsl_v, dissl_v, disf_v, seq_v,
               rown_v, normn_v, deg_sh, dis_sh):
    c = lax.axis_index("c")
    s = lax.axis_index("s")

    drows = npad // 128
    ndt = drows // 8
    eb = erows // NS

    pltpu.sync_copy(colm.at[pl.ds(s * eb, eb)], colb_v)
    pltpu.sync_copy(wm.at[pl.ds(s * eb, eb)], wb_v)
    pltpu.sync_copy(zf.at[pl.ds(0, drows)], deg_v)
    pltpu.sync_copy(seq, seq_v)

    @pl.when(s < ndt)
    def _():
        pltpu.sync_copy(zf.at[pl.ds(0, 8)], deg_sh.at[pl.ds(s * 8, 8)])

    def deg_body(i, _):
        r = i >> 3
        q = (i & 7) * LN
        cols = colb_v[r, pl.ds(q, LN)]
        ws = wb_v[r, pl.ds(q, LN)]
        plsc.addupdate_scatter(deg_v, [cols >> 7, cols & 127], ws)
        return 0

    lax.fori_loop(0, eb * 8, deg_body, 0)
    plsc.subcore_barrier()
    pltpu.sync_copy(deg_v, deg_sh.at[seq_v.at[0]], add=True)
    plsc.subcore_barrier()

    @pl.when(s < ndt)
    def _():
        pltpu.sync_copy(deg_sh.at[pl.ds(s * 8, 8)], degsl_v)

        def dis_body(i, _):
            r = i >> 3
            q = (i & 7) * LN
            dissl_v[r, pl.ds(q, LN)] = _rsqrt_newton(degsl_v[r, pl.ds(q, LN)])
            return 0

        lax.fori_loop(0, 64, dis_body, 0)
        pltpu.sync_copy(dissl_v, dis_sh.at[pl.ds(s * 8, 8)])

    plsc.subcore_barrier()
    pltpu.sync_copy(dis_sh, disf_v)

    pltpu.sync_copy(rowm.at[pl.ds(s * eb, eb)], rown_v)

    def nrm_body(i, _):
        r = i >> 3
        q = (i & 7) * LN
        rr = rown_v[r, pl.ds(q, LN)]
        cc = colb_v[r, pl.ds(q, LN)]
        ww = wb_v[r, pl.ds(q, LN)]
        gr = plsc.load_gather(disf_v, [rr >> 7, rr & 127])
        gc = plsc.load_gather(disf_v, [cc >> 7, cc & 127])
        normn_v[r, pl.ds(q, LN)] = gr * ww * gc
        return 0

    lax.fori_loop(0, eb * 8, nrm_body, 0)

    @pl.when(c == 0)
    def _():
        pltpu.sync_copy(normn_v, norm_out.at[pl.ds(s * eb, eb)])


def _norm_kernel(npad, erows, rowm, colm, wm, zf, seq):
    eb = erows // NS
    return pl.kernel(
        functools.partial(_norm_body, npad, erows),
        out_type=jax.ShapeDtypeStruct((erows, 128), jnp.float32),
        mesh=_mesh(),
        compiler_params=_SC_PARAMS,
        scratch_types=[
            pltpu.VMEM((eb, 128), jnp.int32),
            pltpu.VMEM((eb, 128), jnp.float32),
            pltpu.VMEM((npad // 128, 128), jnp.float32),
            pltpu.VMEM((8, 128), jnp.float32),
            pltpu.VMEM((8, 128), jnp.float32),
            pltpu.VMEM((npad // 128, 128), jnp.float32),
            pltpu.VMEM((1, npad // 128), jnp.int32),
            pltpu.VMEM((eb, 128), jnp.int32),
            pltpu.VMEM((eb, 128), jnp.float32),
            pltpu.VMEM_SHARED((npad // 128, 128), jnp.float32),
            pltpu.VMEM_SHARED((npad // 128, 128), jnp.float32),
        ],
    )(rowm, colm, wm, zf, seq)


def _agg_body(npad, erows, hw2, rowm, colm, normm, out,
              rowb_v, colb_v, normb_v, rows0_v, rows1_v,
              semg0, semg1, sems0, sems1, out_sh):
    c = lax.axis_index("c")
    s = lax.axis_index("s")
    eb = erows // NS
    nsl = npad // NS
    bufs = (rows0_v, rows1_v)
    semg = (semg0, semg1)
    sems = (sems0, sems1)

    zero16 = jnp.zeros((LN,), jnp.float32)

    def z_body(i, _):
        r = i >> 3
        q = (i & 7) * LN
        rows0_v[r, pl.ds(q, LN)] = zero16
        return 0

    lax.fori_loop(0, 1024, z_body, 0)
    for t in range(nsl // 128):
        pltpu.sync_copy(rows0_v, out_sh.at[pl.ds(s * nsl + t * 128, 128)])
    plsc.subcore_barrier()

    off16 = jnp.full((LN,), c * npad, jnp.int32)

    def scale(buf, j):
        def edge_body(e, _):
            nrm = plsc.load_gather(
                normb_v,
                [jnp.full((LN,), j, jnp.int32),
                 jnp.full((LN,), e, jnp.int32)],
            )
            for k in range(8):
                buf[e, pl.ds(k * LN, LN)] = buf[e, pl.ds(k * LN, LN)] * nrm
            return 0

        lax.fori_loop(0, 128, edge_body, 0, unroll=4)

    def super_body(jj, _):
        base = s * eb + jj * 8
        pltpu.sync_copy(rowm.at[pl.ds(base, 8)], rowb_v)
        pltpu.sync_copy(colm.at[pl.ds(base, 8)], colb_v)
        pltpu.sync_copy(normm.at[pl.ds(base, 8)], normb_v)

        def o_body(i, _):
            r = i >> 3
            q = (i & 7) * LN
            rowb_v[r, pl.ds(q, LN)] = rowb_v[r, pl.ds(q, LN)] + off16
            return 0

        lax.fori_loop(0, 64, o_body, 0)

        gat = [None] * 8
        sca = [None] * 8
        gat[0] = pltpu.async_copy(hw2.at[rowb_v.at[0]], bufs[0], semg[0])
        for j in range(8):
            p = j & 1
            gat[j].wait()
            if j + 1 < 8:
                if j - 1 >= 0:
                    sca[j - 1].wait()
                gat[j + 1] = pltpu.async_copy(
                    hw2.at[rowb_v.at[j + 1]], bufs[1 - p], semg[1 - p])
            scale(bufs[p], j)
            sca[j] = pltpu.async_copy(
                bufs[p], out_sh.at[colb_v.at[j]], sems[p], add=True)
        sca[6].wait()
        sca[7].wait()
        return 0

    lax.fori_loop(0, eb // 8, super_body, 0)
    plsc.subcore_barrier()
    pltpu.sync_copy(out_sh.at[pl.ds(s * nsl, nsl)],
                    out.at[c, pl.ds(s * nsl, nsl)])


def _agg_kernel(npad, erows, hw2, rowm, colm, normm):
    return pl.kernel(
        functools.partial(_agg_body, npad, erows),
        out_type=jax.ShapeDtypeStruct((NC, npad, 128), jnp.float32),
        mesh=_mesh(),
        compiler_params=_SC_PARAMS,
        scratch_types=[
            pltpu.VMEM((8, 128), jnp.int32),
            pltpu.VMEM((8, 128), jnp.int32),
            pltpu.VMEM((8, 128), jnp.float32),
            pltpu.VMEM((128, 128), jnp.float32),
            pltpu.VMEM((128, 128), jnp.float32),
            pltpu.SemaphoreType.DMA,
            pltpu.SemaphoreType.DMA,
            pltpu.SemaphoreType.DMA,
            pltpu.SemaphoreType.DMA,
            pltpu.VMEM_SHARED((npad, 128), jnp.float32),
        ],
    )(hw2, rowm, colm, normm)


def _mm_body(apply_act, g_ref, w_ref, b_ref, o_ref):
    h = jnp.concatenate([g_ref[0], g_ref[1]], axis=1)
    if apply_act:
        h = _gelu(h + b_ref[...])
    y = lax.dot_general(h, w_ref[...], (((1,), (1,)), ((), ())),
                        preferred_element_type=jnp.float32)
    o_ref[0] = y[:, :128]
    o_ref[1] = y[:, 128:]


def _mm(npad, g, w, b, apply_act):
    return pl.pallas_call(
        functools.partial(_mm_body, apply_act),
        grid=(npad // 256,),
        in_specs=[
            pl.BlockSpec((2, 256, 128), lambda i: (0, i, 0)),
            pl.BlockSpec((256, 256), lambda i: (0, 0)),
            pl.BlockSpec((1, 256), lambda i: (0, 0)),
        ],
        out_specs=pl.BlockSpec((2, 256, 128), lambda i: (0, i, 0)),
        out_shape=jax.ShapeDtypeStruct((2, npad, 128), jnp.float32),
    )(g, w, b)


def _act_body(g_ref, b_ref, o_ref):
    h = jnp.concatenate([g_ref[0], g_ref[1]], axis=1)
    o_ref[...] = _gelu(h + b_ref[...])


def _act(npad, g, b):
    return pl.pallas_call(
        _act_body,
        grid=(npad // 256,),
        in_specs=[
            pl.BlockSpec((2, 256, 128), lambda i: (0, i, 0)),
            pl.BlockSpec((1, 256), lambda i: (0, 0)),
        ],
        out_specs=pl.BlockSpec((256, 256), lambda i: (i, 0)),
        out_shape=jax.ShapeDtypeStruct((npad, 256), jnp.float32),
    )(g, b)


def kernel(x, edge_index, edge_attr, Ws, bs):
    n, cdim = x.shape
    e = edge_index.shape[1]
    nlayers = Ws.shape[0]
    assert cdim == 256

    npad = -(-n // 2048) * 2048
    ef = e + n
    epad = -(-ef // (NS * 128 * 8)) * (NS * 128 * 8)
    erows = epad // 128

    loop = jnp.arange(n, dtype=edge_index.dtype)
    row_f = jnp.concatenate([edge_index[0], loop])
    col_f = jnp.concatenate([edge_index[1], loop])
    w_f = jnp.concatenate([edge_attr, jnp.ones((n,), x.dtype)])
    perm = jnp.argsort(row_f)
    row_f = row_f[perm]
    col_f = col_f[perm]
    w_f = w_f[perm]
    pad = epad - ef
    rowm = jnp.pad(row_f, (0, pad)).reshape(erows, 128).astype(jnp.int32)
    colm = jnp.pad(col_f, (0, pad)).reshape(erows, 128).astype(jnp.int32)
    wm = jnp.pad(w_f, (0, pad)).reshape(erows, 128)
    seq = jnp.arange(npad // 128, dtype=jnp.int32).reshape(1, npad // 128)
    zf = jnp.zeros((npad // NS, 128), jnp.float32)

    normm = _norm_kernel(npad, erows, rowm, colm, wm, zf, seq)

    xp = jnp.pad(x, ((0, npad - n), (0, 0)))
    g = jnp.stack([xp[:, :128], xp[:, 128:]])
    bz = jnp.zeros((1, cdim), jnp.float32)
    for i in range(nlayers):
        b_prev = bs[i - 1].reshape(1, cdim) if i > 0 else bz
        hw = _mm(npad, g, Ws[i], b_prev, apply_act=(i > 0))
        g = _agg_kernel(npad, erows, hw.reshape(NC * npad, 128),
                        rowm, colm, normm)
    out = _act(npad, g, bs[nlayers - 1].reshape(1, cdim))
    return out[:n]
